# baseline (speedup 1.0000x reference)
import functools

import jax
import jax.numpy as jnp
from jax import lax
from jax.experimental import pallas as pl
from jax.experimental.pallas import tpu as pltpu
from jax.experimental.pallas import tpu_sc as plsc

_IN = 4096
_OUT = 4096
_GS = 8
_ROWS = _OUT * (_IN // _GS)

_NW = 32
_JK = 8
_CH = _JK * 128
_NCHUNK = 4
_OPC = _OUT // _NCHUNK
_RPC = _ROWS // _NCHUNK
_BPW = _RPC // _NW
_NG = _BPW // _CH


def _dequant(table, idx):
    mesh = plsc.VectorSubcoreMesh(core_axis_name="c", subcore_axis_name="s")

    @functools.partial(
        pl.kernel,
        out_type=jax.ShapeDtypeStruct((_RPC, _GS), jnp.float32),
        mesh=mesh,
        compiler_params=pltpu.CompilerParams(use_tc_tiling_on_sc=False),
        scratch_types=[
            pltpu.VMEM((2, _JK, 128), jnp.int32),
            pltpu.VMEM((2, _CH, _GS), jnp.float32),
            pltpu.SemaphoreType.DMA,
            pltpu.SemaphoreType.DMA,
            pltpu.SemaphoreType.DMA,
        ],
    )
    def k(table_hbm, idx_hbm, w_hbm, idx_v, rows_v, sem_i, sem_g, sem_o):
        cid = lax.axis_index("c")
        sid = lax.axis_index("s")
        wid = sid * 2 + cid

        def idx_copy(g, b):
            return pltpu.make_async_copy(
                idx_hbm.at[wid, pl.ds(g * _JK, _JK)], idx_v.at[b], sem_i)

        def out_copy(g, b):
            return pltpu.make_async_copy(
                rows_v.at[b], w_hbm.at[pl.ds(wid * _BPW + g * _CH, _CH)],
                sem_o)

        idx_copy(0, 0).start()

        @pl.loop(0, _NG, step=2)
        def _steps(g0):
            for b in range(2):
                g = g0 + b
                idx_copy(g, b).wait()

                @pl.when(g + 1 < _NG)
                def _():
                    idx_copy(g + 1, 1 - b).start()

                @pl.when(g >= 2)
                def _():
                    out_copy(g - 2, b).wait()

                cps = []
                for j in range(_JK):
                    cps.append(
                        pltpu.async_copy(
                            table_hbm.at[idx_v.at[b].at[j]],
                            rows_v.at[b].at[pl.ds(j * 128, 128)],
                            sem_g,
                        )
                    )
                for cp in cps:
                    cp.wait()
                out_copy(g, b).start()

        out_copy(_NG - 2, 0).wait()
        out_copy(_NG - 1, 1).wait()

    return k(table, idx)


_BM, _BN = 1024, 512


def _mm_chunk(xf, wc, s2d, prev, c):
    m, k = xf.shape
    jbase = (c * _OPC) // _BN

    def mm(x_ref, w_ref, s_ref, *rest):
        o_ref = rest[-1]
        xs = (x_ref[...] * s_ref[...]).astype(jnp.bfloat16)
        wt = w_ref[...].astype(jnp.bfloat16)
        acc = lax.dot_general(
            xs, wt, (((1,), (1,)), ((), ())),
            preferred_element_type=jnp.float32,
        )
        o_ref[...] = jnp.clip(acc, -50.0, 50.0)

    in_specs = [
        pl.BlockSpec((_BM, k), lambda i, j: (i, 0)),
        pl.BlockSpec((_BN, k), lambda i, j: (j, 0)),
        pl.BlockSpec((1, k), lambda i, j: (0, 0)),
    ]
    args = [xf, wc, s2d]
    aliases = {}
    if prev is not None:
        in_specs.append(pl.BlockSpec((8, 128), lambda i, j: (0, 0)))
        args.append(prev)
        aliases = {3: 0}
    return pl.pallas_call(
        mm,
        grid=(m // _BM, _OPC // _BN),
        in_specs=in_specs,
        out_specs=pl.BlockSpec((_BM, _BN), lambda i, j: (i, jbase + j)),
        out_shape=jax.ShapeDtypeStruct((m, _OUT), jnp.float32),
        input_output_aliases=aliases,
    )(*args)


def kernel(x, indices, codebooks, scales):
    table = codebooks[0]
    idx = indices.reshape(_NCHUNK, _NW, _NG * _JK, 128)
    xf = x.reshape(-1, _IN)
    s2d = scales.reshape(1, _IN)
    out = None
    for c in range(_NCHUNK):
        wc = _dequant(table, idx[c]).reshape(_OPC, _IN)
        out = _mm_chunk(xf, wc, s2d, out, c)
    return out.reshape(x.shape[:-1] + (_OUT,))


# speedup vs baseline: 1.1081x; 1.1081x over previous
import functools

import jax
import jax.numpy as jnp
from jax import lax
from jax.experimental import pallas as pl
from jax.experimental.pallas import tpu as pltpu
from jax.experimental.pallas import tpu_sc as plsc

_IN = 4096
_OUT = 4096
_GS = 8
_CB = 65536
_ROWS = _OUT * (_IN // _GS)

_NW = 32
_BPW = _ROWS // _NW
_JK = 8
_CH = _JK * 128
_NG = _BPW // _CH


def _dequant(table, idx):
    mesh = plsc.VectorSubcoreMesh(core_axis_name="c", subcore_axis_name="s")

    @functools.partial(
        pl.kernel,
        out_type=jax.ShapeDtypeStruct((_ROWS, _GS), jnp.float32),
        mesh=mesh,
        compiler_params=pltpu.CompilerParams(use_tc_tiling_on_sc=False),
        scratch_types=[
            pltpu.VMEM((2, _JK, 128), jnp.int32),
            pltpu.VMEM((2, _CH, _GS), jnp.float32),
            pltpu.SemaphoreType.DMA,
            pltpu.SemaphoreType.DMA,
            pltpu.SemaphoreType.DMA,
        ],
    )
    def k(table_hbm, idx_hbm, w_hbm, idx_v, rows_v, sem_i, sem_g, sem_o):
        cid = lax.axis_index("c")
        sid = lax.axis_index("s")
        wid = sid * 2 + cid

        def idx_copy(g, b):
            return pltpu.make_async_copy(
                idx_hbm.at[wid, pl.ds(g * _JK, _JK)], idx_v.at[b], sem_i)

        def out_copy(g, b):
            return pltpu.make_async_copy(
                rows_v.at[b], w_hbm.at[pl.ds(wid * _BPW + g * _CH, _CH)],
                sem_o)

        idx_copy(0, 0).start()

        @pl.loop(0, _NG, step=2)
        def _steps(g0):
            for b in range(2):
                g = g0 + b
                idx_copy(g, b).wait()

                @pl.when(g + 1 < _NG)
                def _():
                    idx_copy(g + 1, 1 - b).start()

                @pl.when(g >= 2)
                def _():
                    out_copy(g - 2, b).wait()

                cps = []
                for j in range(_JK):
                    cps.append(
                        pltpu.async_copy(
                            table_hbm.at[idx_v.at[b].at[j]],
                            rows_v.at[b].at[pl.ds(j * 128, 128)],
                            sem_g,
                        )
                    )
                for cp in cps:
                    cp.wait()
                out_copy(g, b).start()

        out_copy(_NG - 2, 0).wait()
        out_copy(_NG - 1, 1).wait()

    return k(table, idx)


def _matmul(xf, w, s2d):
    m, k = xf.shape
    n = w.shape[0]
    bm, bn = 2048, 256

    def mm(x_ref, w_ref, s_ref, o_ref):
        xs = x_ref[...] * s_ref[...]
        wt = w_ref[...].astype(jnp.bfloat16)
        acc = lax.dot_general(
            xs, wt, (((1,), (1,)), ((), ())),
            preferred_element_type=jnp.float32,
        )
        o_ref[...] = jnp.clip(acc, -50.0, 50.0)

    return pl.pallas_call(
        mm,
        grid=(m // bm, n // bn),
        in_specs=[
            pl.BlockSpec((bm, k), lambda i, j: (i, 0)),
            pl.BlockSpec((bn, k), lambda i, j: (j, 0)),
            pl.BlockSpec((1, k), lambda i, j: (0, 0)),
        ],
        out_specs=pl.BlockSpec((bm, bn), lambda i, j: (i, j)),
        out_shape=jax.ShapeDtypeStruct((m, n), jnp.float32),
    )(xf, w, s2d)


def kernel(x, indices, codebooks, scales):
    table = codebooks[0]
    idx = indices.reshape(_NW, _NG * _JK, 128)
    w = _dequant(table, idx).reshape(_OUT, _IN)
    xf = x.reshape(-1, _IN).astype(jnp.bfloat16)
    s2d = scales.reshape(1, _IN).astype(jnp.bfloat16)
    out = _matmul(xf, w, s2d)
    return out.reshape(x.shape[:-1] + (_OUT,))


# speedup vs baseline: 1.2051x; 1.0875x over previous
import functools

import jax
import jax.numpy as jnp
from jax import lax
from jax.experimental import pallas as pl
from jax.experimental.pallas import tpu as pltpu
from jax.experimental.pallas import tpu_sc as plsc

_IN = 4096
_OUT = 4096
_GS = 8
_CB = 65536
_ROWS = _OUT * (_IN // _GS)

_NW = 32
_BPW = _ROWS // _NW
_JK = 16
_CH = _JK * 128
_NG = _BPW // _CH


def _dequant(table, idx):
    mesh = plsc.VectorSubcoreMesh(core_axis_name="c", subcore_axis_name="s")

    @functools.partial(
        pl.kernel,
        out_type=jax.ShapeDtypeStruct((_ROWS, _GS), jnp.float32),
        mesh=mesh,
        compiler_params=pltpu.CompilerParams(use_tc_tiling_on_sc=False),
        scratch_types=[
            pltpu.VMEM((2, _JK, 128), jnp.int32),
            pltpu.VMEM((2, _CH, _GS), jnp.float32),
            pltpu.SemaphoreType.DMA,
            pltpu.SemaphoreType.DMA,
            pltpu.SemaphoreType.DMA,
        ],
    )
    def k(table_hbm, idx_hbm, w_hbm, idx_v, rows_v, sem_i, sem_g, sem_o):
        cid = lax.axis_index("c")
        sid = lax.axis_index("s")
        wid = sid * 2 + cid

        def idx_copy(g, b):
            return pltpu.make_async_copy(
                idx_hbm.at[wid, pl.ds(g * _JK, _JK)], idx_v.at[b], sem_i)

        def out_copy(g, b):
            return pltpu.make_async_copy(
                rows_v.at[b], w_hbm.at[pl.ds(wid * _BPW + g * _CH, _CH)],
                sem_o)

        idx_copy(0, 0).start()

        @pl.loop(0, _NG, step=2)
        def _steps(g0):
            for b in range(2):
                g = g0 + b
                idx_copy(g, b).wait()

                @pl.when(g + 1 < _NG)
                def _():
                    idx_copy(g + 1, 1 - b).start()

                @pl.when(g >= 2)
                def _():
                    out_copy(g - 2, b).wait()

                cps = []
                for j in range(_JK):
                    cps.append(
                        pltpu.async_copy(
                            table_hbm.at[idx_v.at[b].at[j]],
                            rows_v.at[b].at[pl.ds(j * 128, 128)],
                            sem_g,
                        )
                    )
                for cp in cps:
                    cp.wait()
                out_copy(g, b).start()

        out_copy(_NG - 2, 0).wait()
        out_copy(_NG - 1, 1).wait()

    return k(table, idx)


def _matmul(xf, w, s2d):
    m, k = xf.shape
    n = w.shape[0]
    bm, bn = 1024, 512

    def mm(x_ref, w_ref, s_ref, o_ref):
        xs = (x_ref[...] * s_ref[...]).astype(jnp.bfloat16)
        wt = w_ref[...].astype(jnp.bfloat16)
        acc = lax.dot_general(
            xs, wt, (((1,), (1,)), ((), ())),
            preferred_element_type=jnp.float32,
        )
        o_ref[...] = jnp.clip(acc, -50.0, 50.0)

    return pl.pallas_call(
        mm,
        grid=(m // bm, n // bn),
        in_specs=[
            pl.BlockSpec((bm, k), lambda i, j: (i, 0)),
            pl.BlockSpec((bn, k), lambda i, j: (j, 0)),
            pl.BlockSpec((1, k), lambda i, j: (0, 0)),
        ],
        out_specs=pl.BlockSpec((bm, bn), lambda i, j: (i, j)),
        out_shape=jax.ShapeDtypeStruct((m, n), jnp.float32),
    )(xf, w, s2d)


def kernel(x, indices, codebooks, scales):
    table = codebooks[0]
    idx = indices.reshape(_NW, _NG * _JK, 128)
    w = _dequant(table, idx).reshape(_OUT, _IN)
    xf = x.reshape(-1, _IN)
    out = _matmul(xf, w, scales.reshape(1, _IN))
    return out.reshape(x.shape[:-1] + (_OUT,))


# speedup vs baseline: 1.2250x; 1.0165x over previous
import functools

import jax
import jax.numpy as jnp
from jax import lax
from jax.experimental import pallas as pl
from jax.experimental.pallas import tpu as pltpu
from jax.experimental.pallas import tpu_sc as plsc

_IN = 4096
_OUT = 4096
_GS = 8
_CB = 65536
_ROWS = _OUT * (_IN // _GS)

_NW = 32
_BPW = _ROWS // _NW
_JK = 32
_CH = _JK * 128
_NG = _BPW // _CH


def _dequant(table, idx):
    mesh = plsc.VectorSubcoreMesh(core_axis_name="c", subcore_axis_name="s")

    @functools.partial(
        pl.kernel,
        out_type=jax.ShapeDtypeStruct((_ROWS, _GS), jnp.float32),
        mesh=mesh,
        compiler_params=pltpu.CompilerParams(use_tc_tiling_on_sc=False),
        scratch_types=[
            pltpu.VMEM((2, _JK, 128), jnp.int32),
            pltpu.VMEM((2, _CH, _GS), jnp.float32),
            pltpu.SemaphoreType.DMA,
            pltpu.SemaphoreType.DMA,
            pltpu.SemaphoreType.DMA,
        ],
    )
    def k(table_hbm, idx_hbm, w_hbm, idx_v, rows_v, sem_i, sem_g, sem_o):
        cid = lax.axis_index("c")
        sid = lax.axis_index("s")
        wid = sid * 2 + cid

        def idx_copy(g, b):
            return pltpu.make_async_copy(
                idx_hbm.at[wid, pl.ds(g * _JK, _JK)], idx_v.at[b], sem_i)

        def out_copy(g, b):
            return pltpu.make_async_copy(
                rows_v.at[b], w_hbm.at[pl.ds(wid * _BPW + g * _CH, _CH)],
                sem_o)

        idx_copy(0, 0).start()

        @pl.loop(0, _NG, step=2)
        def _steps(g0):
            for b in range(2):
                g = g0 + b
                idx_copy(g, b).wait()

                @pl.when(g + 1 < _NG)
                def _():
                    idx_copy(g + 1, 1 - b).start()

                @pl.when(g >= 2)
                def _():
                    out_copy(g - 2, b).wait()

                cps = []
                for j in range(_JK):
                    cps.append(
                        pltpu.async_copy(
                            table_hbm.at[idx_v.at[b].at[j]],
                            rows_v.at[b].at[pl.ds(j * 128, 128)],
                            sem_g,
                        )
                    )
                for cp in cps:
                    cp.wait()
                out_copy(g, b).start()

        out_copy(_NG - 2, 0).wait()
        out_copy(_NG - 1, 1).wait()

    return k(table, idx)


def _matmul(xf, w, s2d):
    m, k = xf.shape
    n = w.shape[0]
    bm, bn = 1024, 512

    def mm(x_ref, w_ref, s_ref, o_ref):
        xs = (x_ref[...] * s_ref[...]).astype(jnp.bfloat16)
        wt = w_ref[...].astype(jnp.bfloat16)
        acc = lax.dot_general(
            xs, wt, (((1,), (1,)), ((), ())),
            preferred_element_type=jnp.float32,
        )
        o_ref[...] = jnp.clip(acc, -50.0, 50.0)

    return pl.pallas_call(
        mm,
        grid=(m // bm, n // bn),
        in_specs=[
            pl.BlockSpec((bm, k), lambda i, j: (i, 0)),
            pl.BlockSpec((bn, k), lambda i, j: (j, 0)),
            pl.BlockSpec((1, k), lambda i, j: (0, 0)),
        ],
        out_specs=pl.BlockSpec((bm, bn), lambda i, j: (i, j)),
        out_shape=jax.ShapeDtypeStruct((m, n), jnp.float32),
    )(xf, w, s2d)


def kernel(x, indices, codebooks, scales):
    table = codebooks[0]
    idx = indices.reshape(_NW, _NG * _JK, 128)
    w = _dequant(table, idx).reshape(_OUT, _IN)
    xf = x.reshape(-1, _IN)
    out = _matmul(xf, w, scales.reshape(1, _IN))
    return out.reshape(x.shape[:-1] + (_OUT,))
